# Initial kernel scaffold; baseline (speedup 1.0000x reference)
import jax
import jax.numpy as jnp
from jax.experimental import pallas as pl


def kernel(x, edge_index, W1, b1, W2, b2, fc_W, fc_b):
    raise NotImplementedError("write your pallas kernel here")


# speedup vs baseline: 10.9462x; 10.9462x over previous
import functools

import jax
import jax.numpy as jnp
from jax import lax
from jax.experimental import pallas as pl
from jax.experimental.pallas import tpu as pltpu
from jax.experimental.pallas import tpu_sc as plsc

N = 10000
D = 128
E = 320000
NC = 2
NS = 16
NW = NC * NS
CH = 128
CPT = 79
EPT = CPT * CH
EPAD = EPT * NW
NA = 10240
RPT = NA // NS
RB = RPT // CH
BLK = 1000
GRID = N // BLK


def _mesh():
    return plsc.VectorSubcoreMesh(core_axis_name="c", subcore_axis_name="s",
                                  num_cores=NC, num_subcores=NS)


def _sc_deg(dst_p):
    @functools.partial(
        pl.kernel,
        out_type=jax.ShapeDtypeStruct((NC, NA), jnp.float32),
        mesh=_mesh(),
        scratch_types=[
            pltpu.VMEM((CH,), jnp.int32),
            pltpu.VMEM((CH,), jnp.float32),
            pltpu.VMEM((RPT,), jnp.float32),
            pltpu.VMEM_SHARED((NA,), jnp.float32),
        ],
    )
    def k(dst_hbm, out_hbm, idx_v, val_v, stage_v, acc_sh):
        c = lax.axis_index("c")
        s = lax.axis_index("s")
        for j in range(CH // 16):
            val_v[pl.ds(j * 16, 16)] = jnp.zeros((16,), jnp.float32)
        row0 = s * RPT
        for j in range(RB):
            pltpu.sync_copy(val_v, acc_sh.at[pl.ds(row0 + j * CH, CH)])
        plsc.subcore_barrier()
        for j in range(CH // 16):
            val_v[pl.ds(j * 16, 16)] = jnp.ones((16,), jnp.float32)
        base = (s * NC + c) * EPT

        def body(i, carry):
            pltpu.sync_copy(dst_hbm.at[pl.ds(base + i * CH, CH)], idx_v)
            pltpu.sync_copy(val_v, acc_sh.at[idx_v], add=True)
            return carry

        lax.fori_loop(0, CPT, body, 0)
        plsc.subcore_barrier()
        pltpu.sync_copy(acc_sh.at[pl.ds(row0, RPT)], stage_v)
        pltpu.sync_copy(stage_v, out_hbm.at[c, pl.ds(row0, RPT)])

    return k(dst_p)


def _sc_edge(h, src_p, dst_p):
    @functools.partial(
        pl.kernel,
        out_type=jax.ShapeDtypeStruct((NC, NA, D), jnp.float32),
        mesh=_mesh(),
        scratch_types=[
            pltpu.VMEM((CH,), jnp.int32),
            pltpu.VMEM((CH,), jnp.int32),
            pltpu.VMEM((CH, D), jnp.float32),
            pltpu.VMEM_SHARED((NA, D), jnp.float32),
            pltpu.SemaphoreType.DMA,
        ],
    )
    def k(h_hbm, src_hbm, dst_hbm, out_hbm, sidx_v, didx_v, rows_v, acc_sh,
          sem):
        c = lax.axis_index("c")
        s = lax.axis_index("s")

        def zbody(r, carry):
            for j in range(D // 16):
                rows_v[r, pl.ds(j * 16, 16)] = jnp.zeros((16,), jnp.float32)
            return carry

        lax.fori_loop(0, CH, zbody, 0)
        row0 = s * RPT
        for j in range(RB):
            pltpu.sync_copy(rows_v, acc_sh.at[pl.ds(row0 + j * CH, CH)])
        plsc.subcore_barrier()
        base = (s * NC + c) * EPT

        def body(i, carry):
            off = base + i * CH
            pltpu.sync_copy(src_hbm.at[pl.ds(off, CH)], sidx_v)
            pltpu.sync_copy(dst_hbm.at[pl.ds(off, CH)], didx_v)
            pltpu.async_copy(h_hbm.at[sidx_v], rows_v, sem).wait()
            pltpu.sync_copy(rows_v, acc_sh.at[didx_v], add=True)
            return carry

        lax.fori_loop(0, CPT, body, 0)
        plsc.subcore_barrier()
        for j in range(RB):
            pltpu.sync_copy(acc_sh.at[pl.ds(row0 + j * CH, CH)], rows_v)
            pltpu.sync_copy(rows_v, out_hbm.at[c, pl.ds(row0 + j * CH, CH)])

    return k(h, src_p, dst_p)


def _dinv(d_ref):
    return lax.rsqrt(d_ref[0] + d_ref[1] + 1.0)


def _tc_pre_body(x_ref, w_ref, d_ref, o_ref):
    o_ref[...] = _dinv(d_ref) * jnp.dot(
        x_ref[...], w_ref[...], preferred_element_type=jnp.float32)


def _tc_mid_body(p_ref, ht_ref, w_ref, b_ref, d_ref, o_ref):
    dinv = _dinv(d_ref)
    h1 = jnp.tanh(dinv * (p_ref[0] + p_ref[1] + ht_ref[...]) + b_ref[...])
    o_ref[...] = dinv * jnp.dot(h1, w_ref[...],
                                preferred_element_type=jnp.float32)


def _row_spec(block):
    return pl.BlockSpec(block, lambda i: (i, 0))


def _full_spec(shape):
    ndim = len(shape)
    return pl.BlockSpec(shape, lambda i: (0,) * ndim)


def _tc_pre(x, W, dcol):
    return pl.pallas_call(
        _tc_pre_body,
        grid=(GRID,),
        in_specs=[
            _row_spec((BLK, D)),
            _full_spec((D, D)),
            pl.BlockSpec((NC, BLK, 1), lambda i: (0, i, 0)),
        ],
        out_specs=_row_spec((BLK, D)),
        out_shape=jax.ShapeDtypeStruct((N, D), jnp.float32),
    )(x, W, dcol)


def _tc_mid(p, ht, W, b, dcol):
    return pl.pallas_call(
        _tc_mid_body,
        grid=(GRID,),
        in_specs=[
            pl.BlockSpec((NC, BLK, D), lambda i: (0, i, 0)),
            _row_spec((BLK, D)),
            _full_spec((D, D)),
            _full_spec((1, D)),
            pl.BlockSpec((NC, BLK, 1), lambda i: (0, i, 0)),
        ],
        out_specs=_row_spec((BLK, D)),
        out_shape=jax.ShapeDtypeStruct((N, D), jnp.float32),
    )(p, ht, W, b, dcol)


def _tc_post_body(p_ref, ht_ref, b_ref, fw_ref, fb_ref, d_ref, out_ref,
                  emb_ref):
    dinv = _dinv(d_ref)
    emb = jnp.tanh(dinv * (p_ref[0] + p_ref[1] + ht_ref[...]) + b_ref[...])
    emb_ref[...] = emb
    z = jnp.dot(emb, fw_ref[...], preferred_element_type=jnp.float32)
    out_ref[...] = jax.nn.sigmoid(z + fb_ref[...])


def _tc_post(p, ht, b, fW, fb, dcol):
    return pl.pallas_call(
        _tc_post_body,
        grid=(GRID,),
        in_specs=[
            pl.BlockSpec((NC, BLK, D), lambda i: (0, i, 0)),
            _row_spec((BLK, D)),
            _full_spec((1, D)),
            _full_spec((D, 1)),
            _full_spec((1, 1)),
            pl.BlockSpec((NC, BLK, 1), lambda i: (0, i, 0)),
        ],
        out_specs=[
            _row_spec((BLK, 1)),
            _row_spec((BLK, D)),
        ],
        out_shape=[
            jax.ShapeDtypeStruct((N, 1), jnp.float32),
            jax.ShapeDtypeStruct((N, D), jnp.float32),
        ],
    )(p, ht, b, fW, fb, dcol)


def kernel(x, edge_index, W1, b1, W2, b2, fc_W, fc_b):
    pad = EPAD - E
    src_p = jnp.concatenate([edge_index[0],
                             jnp.zeros((pad,), jnp.int32)])
    dst_p = jnp.concatenate([edge_index[1],
                             jnp.full((pad,), N, jnp.int32)])

    deg_p = _sc_deg(dst_p)
    dcol = deg_p[:, :, None]

    ht1 = _tc_pre(x, W1, dcol)
    p1 = _sc_edge(ht1, src_p, dst_p)
    ht2 = _tc_mid(p1, ht1, W2, b1.reshape(1, D), dcol)
    p2 = _sc_edge(ht2, src_p, dst_p)
    out, emb = _tc_post(p2, ht2, b2.reshape(1, D), fc_W,
                        fc_b.reshape(1, 1), dcol)
    return (out, emb)
